# baseline (speedup 1.0000x reference)
import functools

import jax
import jax.numpy as jnp
from jax import lax
from jax.experimental import pallas as pl
from jax.experimental.pallas import tpu as pltpu
from jax.experimental.pallas import tpu_sc as plsc

D = 128
L = 16
NC, NS = 2, 16
NW = NC * NS
CH = 80
NB = 8


@functools.partial(jax.jit, static_argnames=("E",))
def _dot_scores(table, src, dst, E):
    EPW = E // NW
    NCH = EPW // CH

    mesh = plsc.VectorSubcoreMesh(
        core_axis_name="c", subcore_axis_name="s", num_cores=NC, num_subcores=NS)

    row_bufs = [pltpu.VMEM((CH, D // 2), jnp.int32) for _ in range(2 * NB)]
    sems = [pltpu.SemaphoreType.DMA for _ in range(NB)]

    @functools.partial(
        pl.kernel,
        out_type=jax.ShapeDtypeStruct((E,), jnp.float32),
        mesh=mesh,
        compiler_params=pltpu.CompilerParams(
            needs_layout_passes=False, use_tc_tiling_on_sc=False),
        scratch_types=[
            pltpu.VMEM((EPW,), jnp.int32),
            pltpu.VMEM((EPW,), jnp.int32),
            pltpu.VMEM((EPW + L,), jnp.float32),
        ] + row_bufs + sems,
    )
    def k(table_hbm, src_hbm, dst_hbm, out_hbm, sidx, didx, outv, *bufs_and_sems):
        bufs = [(bufs_and_sems[2 * b], bufs_and_sems[2 * b + 1])
                for b in range(NB)]
        sem = bufs_and_sems[2 * NB:]
        wid = lax.axis_index("s") * NC + lax.axis_index("c")
        base = wid * EPW

        pltpu.sync_copy(src_hbm.at[pl.ds(base, EPW)], sidx)
        pltpu.sync_copy(dst_hbm.at[pl.ds(base, EPW)], didx)

        def fire(c, b):
            sb, db = bufs[b]
            pltpu.async_copy(table_hbm.at[sidx.at[pl.ds(c * CH, CH)]], sb,
                             sem[b])
            pltpu.async_copy(table_hbm.at[didx.at[pl.ds(c * CH, CH)]], db,
                             sem[b])

        def drain(c, b):
            sb, db = bufs[b]
            pltpu.make_async_copy(
                table_hbm.at[sidx.at[pl.ds(c * CH, CH)]], sb, sem[b]).wait()
            pltpu.make_async_copy(
                table_hbm.at[didx.at[pl.ds(c * CH, CH)]], db, sem[b]).wait()

        for b in range(NB):
            fire(b, b)

        def chunk_body(c, carry):
            for b in range(NB):
                @pl.when(lax.rem(c, NB) == b)
                def _(b=b):
                    drain(c, b)
                    sb, db = bufs[b]

                    last_lane = lax.iota(jnp.int32, L) == (L - 1)

                    @plsc.parallel_loop(0, CH, step=1, unroll=4)
                    def edge_body(e):
                        accs = [jnp.zeros((2 * L,), jnp.bfloat16)
                                for _ in range(2)]
                        for j in range(D // (2 * L)):
                            s = plsc.bitcast(sb[e, pl.ds(j * L, L)],
                                             jnp.bfloat16)
                            t = plsc.bitcast(db[e, pl.ds(j * L, L)],
                                             jnp.bfloat16)
                            accs[j % 2] = accs[j % 2] + s * t
                        a0, a1 = plsc.unpack(
                            accs[0] + accs[1],
                            format=plsc.PackFormat.INTERLEAVED)
                        red = plsc.cumsum(a0 + a1)
                        plsc.store_compressed(
                            outv.at[pl.ds(c * CH + e, L)], red, mask=last_lane)

                    @pl.when(c + NB < NCH)
                    def _():
                        fire(c + NB, b)
            return carry

        lax.fori_loop(0, NCH, chunk_body, 0, unroll=False)
        pltpu.sync_copy(outv.at[pl.ds(0, EPW)], out_hbm.at[pl.ds(base, EPW)])

    return k(table, src, dst)


def kernel(node_embeddings, edge_index):
    idx = edge_index.astype(jnp.int32)
    E = idx.shape[1]
    table_bf = node_embeddings.astype(jnp.bfloat16)
    N, Dm = table_bf.shape
    table = lax.bitcast_convert_type(
        table_bf.reshape(N, Dm // 2, 2), jnp.int32)
    scores = _dot_scores(table, idx[0], idx[1], E)
    return scores.reshape(E, 1)


# speedup vs baseline: 1.0296x; 1.0296x over previous
import functools

import jax
import jax.numpy as jnp
from jax import lax
from jax.experimental import pallas as pl
from jax.experimental.pallas import tpu as pltpu
from jax.experimental.pallas import tpu_sc as plsc

D = 128
L = 16
NC, NS = 2, 16
NW = NC * NS
CH = 80
NB = 5


@functools.partial(jax.jit, static_argnames=("E",))
def _dot_scores(table, src, dst, E):
    EPW = E // NW
    NCH = EPW // CH

    mesh = plsc.VectorSubcoreMesh(
        core_axis_name="c", subcore_axis_name="s", num_cores=NC, num_subcores=NS)

    row_bufs = [pltpu.VMEM((CH, D // 2), jnp.int32) for _ in range(2 * NB)]
    sems = [pltpu.SemaphoreType.DMA for _ in range(NB)]

    @functools.partial(
        pl.kernel,
        out_type=jax.ShapeDtypeStruct((E,), jnp.float32),
        mesh=mesh,
        compiler_params=pltpu.CompilerParams(
            needs_layout_passes=False, use_tc_tiling_on_sc=False),
        scratch_types=[
            pltpu.VMEM((EPW,), jnp.int32),
            pltpu.VMEM((EPW,), jnp.int32),
            pltpu.VMEM((EPW + L,), jnp.float32),
        ] + row_bufs + sems,
    )
    def k(table_hbm, src_hbm, dst_hbm, out_hbm, sidx, didx, outv, *bufs_and_sems):
        bufs = [(bufs_and_sems[2 * b], bufs_and_sems[2 * b + 1])
                for b in range(NB)]
        sem = bufs_and_sems[2 * NB:]
        wid = lax.axis_index("s") * NC + lax.axis_index("c")
        base = wid * EPW

        pltpu.sync_copy(src_hbm.at[pl.ds(base, EPW)], sidx)
        pltpu.sync_copy(dst_hbm.at[pl.ds(base, EPW)], didx)

        def fire(c, b):
            sb, db = bufs[b]
            pltpu.async_copy(table_hbm.at[sidx.at[pl.ds(c * CH, CH)]], sb,
                             sem[b])
            pltpu.async_copy(table_hbm.at[didx.at[pl.ds(c * CH, CH)]], db,
                             sem[b])

        def drain(c, b):
            sb, db = bufs[b]
            pltpu.make_async_copy(
                table_hbm.at[sidx.at[pl.ds(c * CH, CH)]], sb, sem[b]).wait()
            pltpu.make_async_copy(
                table_hbm.at[didx.at[pl.ds(c * CH, CH)]], db, sem[b]).wait()

        for b in range(NB):
            fire(b, b)

        def chunk_body(c, carry):
            for b in range(NB):
                @pl.when(lax.rem(c, NB) == b)
                def _(b=b):
                    drain(c, b)
                    sb, db = bufs[b]

                    last_lane = lax.iota(jnp.int32, L) == (L - 1)

                    @plsc.parallel_loop(0, CH, step=1, unroll=4)
                    def edge_body(e):
                        accs = [jnp.zeros((2 * L,), jnp.bfloat16)
                                for _ in range(2)]
                        for j in range(D // (2 * L)):
                            s = plsc.bitcast(sb[e, pl.ds(j * L, L)],
                                             jnp.bfloat16)
                            t = plsc.bitcast(db[e, pl.ds(j * L, L)],
                                             jnp.bfloat16)
                            accs[j % 2] = accs[j % 2] + s * t
                        a0, a1 = plsc.unpack(
                            accs[0] + accs[1],
                            format=plsc.PackFormat.INTERLEAVED)
                        red = plsc.cumsum(a0 + a1)
                        plsc.store_compressed(
                            outv.at[pl.ds(c * CH + e, L)], red, mask=last_lane)

                    @pl.when(c + NB < NCH)
                    def _():
                        fire(c + NB, b)
            return carry

        lax.fori_loop(0, NCH, chunk_body, 0, unroll=False)
        pltpu.sync_copy(outv.at[pl.ds(0, EPW)], out_hbm.at[pl.ds(base, EPW)])

    return k(table, src, dst)


def kernel(node_embeddings, edge_index):
    idx = edge_index.astype(jnp.int32)
    E = idx.shape[1]
    table_bf = node_embeddings.astype(jnp.bfloat16)
    N, Dm = table_bf.shape
    table = lax.bitcast_convert_type(
        table_bf.reshape(N, Dm // 2, 2), jnp.int32)
    scores = _dot_scores(table, idx[0], idx[1], E)
    return scores.reshape(E, 1)
